# baseline (speedup 1.0000x reference)
import math

import jax
import jax.numpy as jnp
from jax.experimental import pallas as pl
from jax.experimental.pallas import tpu as pltpu

HID = 768
NH = 12
HD = HID // NH
FF = 3072
NL = 4
PS = 8
GRID = 8
NPATCH = GRID * GRID * GRID
NTOK = 16
SEQ = NTOK + NPATCH
B = 2
ROWS = B * SEQ
FF_CHUNK = 768
NMAT = 6


def _ln(x, g, b, eps=1e-5):
    m = jnp.mean(x, axis=-1, keepdims=True)
    xc = x - m
    v = jnp.mean(xc * xc, axis=-1, keepdims=True)
    return xc * jax.lax.rsqrt(v + eps) * g + b


def _mm_t(a, w, out_dtype=jnp.float32):
    return jax.lax.dot_general(a.astype(jnp.bfloat16), w.astype(jnp.bfloat16),
                               (((1,), (1,)), ((), ())),
                               preferred_element_type=out_dtype)


def _attention(q, k, v):
    scale = 1.0 / math.sqrt(HD)
    row = jax.lax.broadcasted_iota(jnp.int32, (SEQ, SEQ), 0)
    col = jax.lax.broadcasted_iota(jnp.int32, (SEQ, SEQ), 1)
    causal = row >= col
    outs = []
    for bi in range(B):
        r0 = bi * SEQ
        head_outs = []
        for hi in range(NH):
            c0 = hi * HD
            qb = q[r0:r0 + SEQ, c0:c0 + HD]
            kb = k[r0:r0 + SEQ, c0:c0 + HD]
            vb = v[r0:r0 + SEQ, c0:c0 + HD]
            logits = jax.lax.dot_general(
                qb, kb, (((1,), (1,)), ((), ())),
                preferred_element_type=jnp.float32) * scale
            logits = jnp.where(causal, logits, jnp.float32(-1e9))
            m = jnp.max(logits, axis=-1, keepdims=True)
            p = jnp.exp(logits - m)
            s = jnp.sum(p, axis=-1, keepdims=True)
            attn = (p / s).astype(jnp.bfloat16)
            head_outs.append(
                jnp.dot(attn, vb, preferred_element_type=jnp.float32))
        outs.append(jnp.concatenate(head_outs, axis=1))
    return jnp.concatenate(outs, axis=0)


def _model_kernel(*refs):
    it = iter(refs)
    tok_ref = next(it)
    patch_ref = next(it)
    wp_ref = next(it)
    bp_ref = next(it)
    wmats = [[next(it) for _ in range(NMAT)] for _ in range(NL)]
    small = [[next(it) for _ in range(7)] for _ in range(NL)]
    nfg_ref = next(it)
    nfb_ref = next(it)
    out_ref = next(it)
    bufs = [next(it) for _ in range(NMAT)]
    sems = next(it)

    def copy(l, i):
        return pltpu.make_async_copy(wmats[l][i], bufs[i], sems.at[i])

    for i in range(NMAT):
        copy(0, i).start()

    img = jnp.dot(patch_ref[...].astype(jnp.bfloat16),
                  wp_ref[...].astype(jnp.bfloat16),
                  preferred_element_type=jnp.float32) + bp_ref[...]
    rows = []
    for bi in range(B):
        rows.append(tok_ref[bi * NTOK:(bi + 1) * NTOK, :])
        rows.append(img[bi * NPATCH:(bi + 1) * NPATCH, :])
    x = jnp.concatenate(rows, axis=0)

    for l in range(NL):
        sv = [r[...] for r in small[l]]
        bo, ln1g, ln1b, ln2g, ln2b, b1, b2 = sv
        for i in range(4):
            copy(l, i).wait()
        h = _ln(x, ln1g, ln1b)
        q = _mm_t(h, bufs[0][...]).astype(jnp.bfloat16)
        k = _mm_t(h, bufs[1][...]).astype(jnp.bfloat16)
        v = _mm_t(h, bufs[2][...]).astype(jnp.bfloat16)
        o = _attention(q, k, v)
        x = x + _mm_t(o, bufs[3][...]) + bo
        if l + 1 < NL:
            for i in range(4):
                copy(l + 1, i).start()
        copy(l, 4).wait()
        copy(l, 5).wait()
        h2 = _ln(x, ln2g, ln2b)
        acc = x
        for c in range(0, FF, FF_CHUNK):
            w1c = bufs[4][c:c + FF_CHUNK, :]
            b1c = b1[:, c:c + FF_CHUNK]
            w2c = bufs[5][:, c:c + FF_CHUNK]
            ff = jax.nn.gelu((_mm_t(h2, w1c) + b1c).astype(jnp.bfloat16))
            acc = acc + _mm_t(ff, w2c)
        x = acc + b2
        if l + 1 < NL:
            copy(l + 1, 4).start()
            copy(l + 1, 5).start()

    out_ref[...] = _ln(x, nfg_ref[...], nfb_ref[...])


_HBM_SPEC = pl.BlockSpec(memory_space=pltpu.MemorySpace.HBM)
_VMEM_SPEC = pl.BlockSpec(memory_space=pltpu.MemorySpace.VMEM)


@jax.jit
def _run(input_ids, input_image, params):
    tok = params['embed'][input_ids].reshape(B * NTOK, HID)
    img = input_image.reshape(B, 1, GRID, PS, GRID, PS, GRID, PS)
    patches = img.transpose(0, 2, 4, 6, 1, 3, 5, 7).reshape(
        B * NPATCH, PS * PS * PS)
    layers = params['layers']
    r2 = lambda a: a.reshape(1, -1)

    args = [tok, patches, params['Wp'], r2(params['bp'])]
    specs = [_VMEM_SPEC] * 4
    for lp in layers:
        args += [lp['Wq'], lp['Wk'], lp['Wv'], lp['Wo'], lp['W1'], lp['W2']]
        specs += [_HBM_SPEC] * NMAT
    for lp in layers:
        args += [r2(lp['bo']), r2(lp['ln1_g']), r2(lp['ln1_b']),
                 r2(lp['ln2_g']), r2(lp['ln2_b']), r2(lp['b1']), r2(lp['b2'])]
        specs += [_VMEM_SPEC] * 7
    args += [r2(params['nf_g']), r2(params['nf_b'])]
    specs += [_VMEM_SPEC] * 2

    x = pl.pallas_call(
        _model_kernel,
        out_shape=jax.ShapeDtypeStruct((ROWS, HID), jnp.float32),
        in_specs=specs,
        out_specs=_VMEM_SPEC,
        scratch_shapes=[
            pltpu.VMEM((HID, HID), jnp.float32),
            pltpu.VMEM((HID, HID), jnp.float32),
            pltpu.VMEM((HID, HID), jnp.float32),
            pltpu.VMEM((HID, HID), jnp.float32),
            pltpu.VMEM((FF, HID), jnp.float32),
            pltpu.VMEM((HID, FF), jnp.float32),
            pltpu.SemaphoreType.DMA((NMAT,)),
        ],
        compiler_params=pltpu.CompilerParams(
            vmem_limit_bytes=63 * 1024 * 1024),
    )(*args)
    return x.reshape(B, SEQ, HID)


def kernel(input_ids, input_image, params):
    return _run(input_ids, input_image, params)


# speedup vs baseline: 1.0252x; 1.0252x over previous
import math

import jax
import jax.numpy as jnp
from jax.experimental import pallas as pl
from jax.experimental.pallas import tpu as pltpu

HID = 768
NH = 12
HD = HID // NH
FF = 3072
NL = 4
PS = 8
GRID = 8
NPATCH = GRID * GRID * GRID
NTOK = 16
SEQ = NTOK + NPATCH
B = 2
ROWS = B * SEQ
FF_CHUNK = 768
NMAT = 6


def _ln(x, g, b, eps=1e-5):
    m = jnp.mean(x, axis=-1, keepdims=True)
    xc = x - m
    v = jnp.mean(xc * xc, axis=-1, keepdims=True)
    return xc * jax.lax.rsqrt(v + eps) * g + b


def _mm_t(a, w, out_dtype=jnp.float32):
    return jax.lax.dot_general(a.astype(jnp.bfloat16), w.astype(jnp.bfloat16),
                               (((1,), (1,)), ((), ())),
                               preferred_element_type=out_dtype)


def _attention(q, k, v):
    scale = 1.0 / math.sqrt(HD)
    row = jax.lax.broadcasted_iota(jnp.int32, (SEQ, SEQ), 0)
    col = jax.lax.broadcasted_iota(jnp.int32, (SEQ, SEQ), 1)
    causal = row >= col
    outs = []
    for bi in range(B):
        r0 = bi * SEQ
        head_outs = []
        for hi in range(NH):
            c0 = hi * HD
            qb = q[r0:r0 + SEQ, c0:c0 + HD]
            kb = k[r0:r0 + SEQ, c0:c0 + HD]
            vb = v[r0:r0 + SEQ, c0:c0 + HD]
            logits = (jax.lax.dot_general(
                qb, kb, (((1,), (1,)), ((), ())),
                preferred_element_type=jnp.float32) * scale).astype(
                    jnp.bfloat16)
            logits = jnp.where(causal, logits, jnp.bfloat16(-1e9))
            m = jnp.max(logits, axis=-1, keepdims=True)
            p = jnp.exp(logits - m)
            s = jnp.sum(p, axis=-1, keepdims=True, dtype=jnp.float32)
            attn = p * (1.0 / s).astype(jnp.bfloat16)
            head_outs.append(
                jnp.dot(attn, vb, preferred_element_type=jnp.float32))
        outs.append(jnp.concatenate(head_outs, axis=1))
    return jnp.concatenate(outs, axis=0)


def _model_kernel(*refs):
    it = iter(refs)
    tok_ref = next(it)
    patch_ref = next(it)
    wp_ref = next(it)
    bp_ref = next(it)
    wmats = [[next(it) for _ in range(NMAT)] for _ in range(NL)]
    small = [[next(it) for _ in range(7)] for _ in range(NL)]
    nfg_ref = next(it)
    nfb_ref = next(it)
    out_ref = next(it)
    bufs = [next(it) for _ in range(NMAT)]
    sems = next(it)

    def copy(l, i):
        return pltpu.make_async_copy(wmats[l][i], bufs[i], sems.at[i])

    for i in range(NMAT):
        copy(0, i).start()

    img = jnp.dot(patch_ref[...].astype(jnp.bfloat16),
                  wp_ref[...].astype(jnp.bfloat16),
                  preferred_element_type=jnp.float32) + bp_ref[...]
    rows = []
    for bi in range(B):
        rows.append(tok_ref[bi * NTOK:(bi + 1) * NTOK, :])
        rows.append(img[bi * NPATCH:(bi + 1) * NPATCH, :])
    x = jnp.concatenate(rows, axis=0)

    for l in range(NL):
        sv = [r[...] for r in small[l]]
        bo, ln1g, ln1b, ln2g, ln2b, b1, b2 = sv
        for i in range(4):
            copy(l, i).wait()
        h = _ln(x, ln1g, ln1b)
        q = _mm_t(h, bufs[0][...]).astype(jnp.bfloat16)
        k = _mm_t(h, bufs[1][...]).astype(jnp.bfloat16)
        v = _mm_t(h, bufs[2][...]).astype(jnp.bfloat16)
        o = _attention(q, k, v)
        x = x + _mm_t(o, bufs[3][...]) + bo
        if l + 1 < NL:
            for i in range(4):
                copy(l + 1, i).start()
        copy(l, 4).wait()
        copy(l, 5).wait()
        h2 = _ln(x, ln2g, ln2b)
        acc = x
        for c in range(0, FF, FF_CHUNK):
            w1c = bufs[4][c:c + FF_CHUNK, :]
            b1c = b1[:, c:c + FF_CHUNK]
            w2c = bufs[5][:, c:c + FF_CHUNK]
            ff = jax.nn.gelu((_mm_t(h2, w1c) + b1c).astype(jnp.bfloat16))
            acc = acc + _mm_t(ff, w2c)
        x = acc + b2
        if l + 1 < NL:
            copy(l + 1, 4).start()
            copy(l + 1, 5).start()

    out_ref[...] = _ln(x, nfg_ref[...], nfb_ref[...])


_HBM_SPEC = pl.BlockSpec(memory_space=pltpu.MemorySpace.HBM)
_VMEM_SPEC = pl.BlockSpec(memory_space=pltpu.MemorySpace.VMEM)


@jax.jit
def _run(input_ids, input_image, params):
    tok = params['embed'][input_ids].reshape(B * NTOK, HID)
    img = input_image.reshape(B, 1, GRID, PS, GRID, PS, GRID, PS)
    patches = img.transpose(0, 2, 4, 6, 1, 3, 5, 7).reshape(
        B * NPATCH, PS * PS * PS)
    layers = params['layers']
    r2 = lambda a: a.reshape(1, -1)

    args = [tok, patches, params['Wp'], r2(params['bp'])]
    specs = [_VMEM_SPEC] * 4
    for lp in layers:
        args += [lp['Wq'], lp['Wk'], lp['Wv'], lp['Wo'], lp['W1'], lp['W2']]
        specs += [_HBM_SPEC] * NMAT
    for lp in layers:
        args += [r2(lp['bo']), r2(lp['ln1_g']), r2(lp['ln1_b']),
                 r2(lp['ln2_g']), r2(lp['ln2_b']), r2(lp['b1']), r2(lp['b2'])]
        specs += [_VMEM_SPEC] * 7
    args += [r2(params['nf_g']), r2(params['nf_b'])]
    specs += [_VMEM_SPEC] * 2

    x = pl.pallas_call(
        _model_kernel,
        out_shape=jax.ShapeDtypeStruct((ROWS, HID), jnp.float32),
        in_specs=specs,
        out_specs=_VMEM_SPEC,
        scratch_shapes=[
            pltpu.VMEM((HID, HID), jnp.float32),
            pltpu.VMEM((HID, HID), jnp.float32),
            pltpu.VMEM((HID, HID), jnp.float32),
            pltpu.VMEM((HID, HID), jnp.float32),
            pltpu.VMEM((FF, HID), jnp.float32),
            pltpu.VMEM((HID, FF), jnp.float32),
            pltpu.SemaphoreType.DMA((NMAT,)),
        ],
        compiler_params=pltpu.CompilerParams(
            vmem_limit_bytes=63 * 1024 * 1024),
    )(*args)
    return x.reshape(B, SEQ, HID)


def kernel(input_ids, input_image, params):
    return _run(input_ids, input_image, params)
